# baseline (speedup 1.0000x reference)
import functools

import jax
import jax.numpy as jnp
import numpy as np
from jax import lax
from jax.experimental import pallas as pl
from jax.experimental.pallas import tpu as pltpu
from jax.experimental.pallas import tpu_sc as plsc

_TOKENS = 32768
_D = 512
_KEEP_PROB = 0.9

_BLK = 4096


def _rotl(x, d):
    return ((x << np.uint32(d)) | (x >> np.uint32(32 - d))).astype(np.uint32)


def _threefry2x32(k1, k2, x0, x1):
    rot = [np.uint32(r) for r in (13, 15, 26, 6, 17, 29, 16, 24)]
    r0, r1 = rot[:4], rot[4:]
    ks0, ks1 = np.uint32(k1), np.uint32(k2)
    ks2 = ks0 ^ ks1 ^ np.uint32(0x1BD11BDA)
    x0 = (x0 + ks0).astype(np.uint32)
    x1 = (x1 + ks1).astype(np.uint32)

    def rounds(x0, x1, rots):
        for r in rots:
            x0 = (x0 + x1).astype(np.uint32)
            x1 = _rotl(x1, r) ^ x0
        return x0, x1

    x0, x1 = rounds(x0, x1, r0)
    x0 = (x0 + ks1).astype(np.uint32)
    x1 = (x1 + ks2 + np.uint32(1)).astype(np.uint32)
    x0, x1 = rounds(x0, x1, r1)
    x0 = (x0 + ks2).astype(np.uint32)
    x1 = (x1 + ks0 + np.uint32(2)).astype(np.uint32)
    x0, x1 = rounds(x0, x1, r0)
    x0 = (x0 + ks0).astype(np.uint32)
    x1 = (x1 + ks1 + np.uint32(3)).astype(np.uint32)
    x0, x1 = rounds(x0, x1, r1)
    x0 = (x0 + ks1).astype(np.uint32)
    x1 = (x1 + ks2 + np.uint32(4)).astype(np.uint32)
    x0, x1 = rounds(x0, x1, r0)
    x0 = (x0 + ks2).astype(np.uint32)
    x1 = (x1 + ks0 + np.uint32(5)).astype(np.uint32)
    return x0, x1


def _dropout_mask():
    n, seed = _TOKENS, 42
    if jax.config.jax_threefry_partitionable:
        y0, y1 = _threefry2x32(0, seed, np.zeros(n, np.uint32),
                               np.arange(n, dtype=np.uint32))
        bits = y0 ^ y1
    else:
        cnt = np.arange(n, dtype=np.uint32)
        y0, y1 = _threefry2x32(0, seed, cnt[: n // 2], cnt[n // 2:])
        bits = np.concatenate([y0, y1])
    fb = (bits >> np.uint32(9)) | np.uint32(0x3F800000)
    u = fb.view(np.float32) - np.float32(1.0)
    return u < np.float32(_KEEP_PROB)


def _mask_body(x_ref, m_ref, o_ref):
    x = x_ref[...].reshape(_BLK // 128, 128, _D)
    m = m_ref[...].reshape(_BLK // 128, 128, 1)
    o_ref[...] = (x * m).reshape(_BLK, _D)


def _kernel_tc(flat):
    mask = jnp.asarray(
        _dropout_mask().astype(np.float32).reshape(_TOKENS // 128, 128))
    grid = _TOKENS // _BLK
    return pl.pallas_call(
        _mask_body,
        grid=(grid,),
        in_specs=[
            pl.BlockSpec((_BLK, _D), lambda i: (i, 0)),
            pl.BlockSpec((_BLK // 128, 128), lambda i: (i, 0)),
        ],
        out_specs=pl.BlockSpec((_BLK, _D), lambda i: (i, 0)),
        out_shape=jax.ShapeDtypeStruct((_TOKENS, _D), jnp.float32),
        compiler_params=pltpu.CompilerParams(
            vmem_limit_bytes=128 * 1024 * 1024),
    )(flat, mask)


_NC, _NS = 2, 16
_NW = _NC * _NS
_CK = 32
_CD = 16
_RING = 4


def _pad_split(idx, chunk):
    per_w = -(-len(idx) // (_NW * chunk)) * chunk
    total = per_w * _NW
    pad = np.resize(idx[-1:], total - len(idx)) if total > len(idx) else idx[:0]
    return np.concatenate([idx, pad]).astype(np.int32).reshape(_NW, per_w // chunk, chunk)


def _kernel_sc(flat):
    mask = _dropout_mask()
    kept3 = _pad_split(np.flatnonzero(mask), _CK)
    drop3 = _pad_split(np.flatnonzero(~mask), _CD)
    nch, ndch = kept3.shape[1], drop3.shape[1]

    mesh = plsc.VectorSubcoreMesh(core_axis_name="c", subcore_axis_name="s")

    @functools.partial(
        pl.kernel,
        out_type=jax.ShapeDtypeStruct((_TOKENS, _D), jnp.float32),
        mesh=mesh,
        scratch_types=(
            [pltpu.VMEM((nch, _CK), jnp.int32),
             pltpu.VMEM((ndch, _CD), jnp.int32),
             pltpu.VMEM((_CD, _D), jnp.float32)]
            + [pltpu.VMEM((_CK, _D), jnp.float32) for _ in range(_RING)]
            + [pltpu.SemaphoreType.DMA for _ in range(2 * _RING + 1)]
        ),
    )
    def body(flat_hbm, kidx_hbm, didx_hbm, out_hbm, kidx_v, didx_v, zbuf,
             *bufs_and_sems):
        bufs = bufs_and_sems[:_RING]
        gsem = bufs_and_sems[_RING:2 * _RING]
        ssem = bufs_and_sems[2 * _RING:3 * _RING]
        zsem = bufs_and_sems[3 * _RING]
        wid = lax.axis_index("s") * _NC + lax.axis_index("c")

        pltpu.sync_copy(kidx_hbm.at[wid], kidx_v)
        pltpu.sync_copy(didx_hbm.at[wid], didx_v)

        zero = jnp.zeros((16,), jnp.float32)
        for r in range(_CD):
            for k in range(_D // 16):
                zbuf[r, pl.ds(k * 16, 16)] = zero
        zh = [pltpu.async_copy(zbuf, out_hbm.at[didx_v.at[j]], zsem)
              for j in range(ndch)]

        g = [None] * nch
        s = [None] * nch
        waited = [False] * nch
        for j in range(nch + 1):
            if j < nch:
                b = j % _RING
                if j >= _RING:
                    s[j - _RING].wait()
                    waited[j - _RING] = True
                g[j] = pltpu.async_copy(flat_hbm.at[kidx_v.at[j]], bufs[b], gsem[b])
            if j >= 1:
                i = j - 1
                g[i].wait()
                s[i] = pltpu.async_copy(bufs[i % _RING], out_hbm.at[kidx_v.at[i]],
                                        ssem[i % _RING])
        for i in range(nch):
            if not waited[i]:
                s[i].wait()
        for h in zh:
            h.wait()

    return body(flat, jnp.asarray(kept3), jnp.asarray(drop3))


_RC = 64
_LRING = 3


def _pad_split_by_range(idx, chunk, rows_w):
    per_w = [idx[(idx >= w * rows_w) & (idx < (w + 1) * rows_w)] for w in range(_NW)]
    assert all(len(p) > 0 for p in per_w)
    ndch = max(-(-len(p) // chunk) for p in per_w)
    out = np.empty((_NW, ndch * chunk), np.int32)
    for w, p in enumerate(per_w):
        out[w, :len(p)] = p
        out[w, len(p):] = p[-1]
    return out.reshape(_NW, ndch, chunk)


def _kernel_sc_linear(flat):
    mask = _dropout_mask()
    rows_w = _TOKENS // _NW
    drop3 = _pad_split_by_range(np.flatnonzero(~mask), _CD, rows_w)
    ndch = drop3.shape[1]
    nch = rows_w // _RC

    mesh = plsc.VectorSubcoreMesh(core_axis_name="c", subcore_axis_name="s")

    @functools.partial(
        pl.kernel,
        out_type=jax.ShapeDtypeStruct((_TOKENS, _D), jnp.float32),
        mesh=mesh,
        scratch_types=(
            [pltpu.VMEM((ndch, _CD), jnp.int32),
             pltpu.VMEM((_CD, _D), jnp.float32)]
            + [pltpu.VMEM((_RC, _D), jnp.float32) for _ in range(_LRING)]
            + [pltpu.SemaphoreType.DMA for _ in range(2 * _LRING + 1)]
        ),
    )
    def body(flat_hbm, didx_hbm, out_hbm, didx_v, zbuf, *bufs_and_sems):
        bufs = bufs_and_sems[:_LRING]
        gsem = bufs_and_sems[_LRING:2 * _LRING]
        ssem = bufs_and_sems[2 * _LRING:3 * _LRING]
        zsem = bufs_and_sems[3 * _LRING]
        wid = lax.axis_index("s") * _NC + lax.axis_index("c")
        base = wid * rows_w

        pltpu.sync_copy(didx_hbm.at[wid], didx_v)

        zero = jnp.zeros((16,), jnp.float32)
        for r in range(_CD):
            for k in range(_D // 16):
                zbuf[r, pl.ds(k * 16, 16)] = zero

        g = [None] * nch
        s = [None] * nch
        waited = [False] * nch
        for j in range(nch + 1):
            if j < nch:
                b = j % _LRING
                if j >= _LRING:
                    s[j - _LRING].wait()
                    waited[j - _LRING] = True
                g[j] = pltpu.async_copy(
                    flat_hbm.at[pl.ds(base + j * _RC, _RC)], bufs[b], gsem[b])
            if j >= 1:
                i = j - 1
                g[i].wait()
                s[i] = pltpu.async_copy(
                    bufs[i % _LRING], out_hbm.at[pl.ds(base + i * _RC, _RC)],
                    ssem[i % _LRING])
        for i in range(nch):
            if not waited[i]:
                s[i].wait()
        zh = [pltpu.async_copy(zbuf, out_hbm.at[didx_v.at[j]], zsem)
              for j in range(ndch)]
        for h in zh:
            h.wait()

    return body(flat, jnp.asarray(drop3))


_MCHUNK = 1024
_MIN_DEPTH = 6
_MOUT_DEPTH = 4


def _kernel_tc_manual(flat):
    mask = jnp.asarray(
        _dropout_mask().astype(np.float32).reshape(_TOKENS // 128, 128))
    n = _TOKENS // _MCHUNK
    mrows = _MCHUNK // 128

    def body(x_hbm, m_ref, o_hbm, ibufs, obufs, isems, osems):
        def copy_in(j):
            return pltpu.make_async_copy(
                x_hbm.at[pl.ds(j * _MCHUNK, _MCHUNK)],
                ibufs.at[j % _MIN_DEPTH], isems.at[j % _MIN_DEPTH])

        def copy_out(j):
            return pltpu.make_async_copy(
                obufs.at[j % _MOUT_DEPTH],
                o_hbm.at[pl.ds(j * _MCHUNK, _MCHUNK)], osems.at[j % _MOUT_DEPTH])

        for k in range(min(_MIN_DEPTH, n)):
            copy_in(k).start()
        for j in range(n):
            copy_in(j).wait()
            if j >= _MOUT_DEPTH:
                copy_out(j - _MOUT_DEPTH).wait()
            x = ibufs[j % _MIN_DEPTH].reshape(mrows, 128, _D)
            m = m_ref[pl.ds(j * mrows, mrows), :].reshape(mrows, 128, 1)
            obufs[j % _MOUT_DEPTH] = (x * m).reshape(_MCHUNK, _D)
            copy_out(j).start()
            nk = j + _MIN_DEPTH
            if nk < n:
                copy_in(nk).start()
        for j in range(max(0, n - _MOUT_DEPTH), n):
            copy_out(j).wait()

    return pl.pallas_call(
        body,
        in_specs=[
            pl.BlockSpec(memory_space=pl.ANY),
            pl.BlockSpec(memory_space=pltpu.VMEM),
        ],
        out_specs=pl.BlockSpec(memory_space=pl.ANY),
        out_shape=jax.ShapeDtypeStruct((_TOKENS, _D), jnp.float32),
        scratch_shapes=[
            pltpu.VMEM((_MIN_DEPTH, _MCHUNK, _D), jnp.float32),
            pltpu.VMEM((_MOUT_DEPTH, _MCHUNK, _D), jnp.float32),
            pltpu.SemaphoreType.DMA((_MIN_DEPTH,)),
            pltpu.SemaphoreType.DMA((_MOUT_DEPTH,)),
        ],
        compiler_params=pltpu.CompilerParams(
            vmem_limit_bytes=60 * 1024 * 1024),
    )(flat, mask)


def kernel(flat, row_starts):
    del row_starts
    return _kernel_tc_manual(flat)


# speedup vs baseline: 1.0112x; 1.0112x over previous
import functools

import jax
import jax.numpy as jnp
import numpy as np
from jax import lax
from jax.experimental import pallas as pl
from jax.experimental.pallas import tpu as pltpu
from jax.experimental.pallas import tpu_sc as plsc

_TOKENS = 32768
_D = 512
_KEEP_PROB = 0.9

_BLK = 4096


def _rotl(x, d):
    return ((x << np.uint32(d)) | (x >> np.uint32(32 - d))).astype(np.uint32)


def _threefry2x32(k1, k2, x0, x1):
    rot = [np.uint32(r) for r in (13, 15, 26, 6, 17, 29, 16, 24)]
    r0, r1 = rot[:4], rot[4:]
    ks0, ks1 = np.uint32(k1), np.uint32(k2)
    ks2 = ks0 ^ ks1 ^ np.uint32(0x1BD11BDA)
    x0 = (x0 + ks0).astype(np.uint32)
    x1 = (x1 + ks1).astype(np.uint32)

    def rounds(x0, x1, rots):
        for r in rots:
            x0 = (x0 + x1).astype(np.uint32)
            x1 = _rotl(x1, r) ^ x0
        return x0, x1

    x0, x1 = rounds(x0, x1, r0)
    x0 = (x0 + ks1).astype(np.uint32)
    x1 = (x1 + ks2 + np.uint32(1)).astype(np.uint32)
    x0, x1 = rounds(x0, x1, r1)
    x0 = (x0 + ks2).astype(np.uint32)
    x1 = (x1 + ks0 + np.uint32(2)).astype(np.uint32)
    x0, x1 = rounds(x0, x1, r0)
    x0 = (x0 + ks0).astype(np.uint32)
    x1 = (x1 + ks1 + np.uint32(3)).astype(np.uint32)
    x0, x1 = rounds(x0, x1, r1)
    x0 = (x0 + ks1).astype(np.uint32)
    x1 = (x1 + ks2 + np.uint32(4)).astype(np.uint32)
    x0, x1 = rounds(x0, x1, r0)
    x0 = (x0 + ks2).astype(np.uint32)
    x1 = (x1 + ks0 + np.uint32(5)).astype(np.uint32)
    return x0, x1


def _dropout_mask():
    n, seed = _TOKENS, 42
    if jax.config.jax_threefry_partitionable:
        y0, y1 = _threefry2x32(0, seed, np.zeros(n, np.uint32),
                               np.arange(n, dtype=np.uint32))
        bits = y0 ^ y1
    else:
        cnt = np.arange(n, dtype=np.uint32)
        y0, y1 = _threefry2x32(0, seed, cnt[: n // 2], cnt[n // 2:])
        bits = np.concatenate([y0, y1])
    fb = (bits >> np.uint32(9)) | np.uint32(0x3F800000)
    u = fb.view(np.float32) - np.float32(1.0)
    return u < np.float32(_KEEP_PROB)


def _mask_body(x_ref, m_ref, o_ref):
    x = x_ref[...].reshape(_BLK // 128, 128, _D)
    m = m_ref[...].reshape(_BLK // 128, 128, 1)
    o_ref[...] = (x * m).reshape(_BLK, _D)


def _kernel_tc(flat):
    mask = jnp.asarray(
        _dropout_mask().astype(np.float32).reshape(_TOKENS // 128, 128))
    grid = _TOKENS // _BLK
    return pl.pallas_call(
        _mask_body,
        grid=(grid,),
        in_specs=[
            pl.BlockSpec((_BLK, _D), lambda i: (i, 0)),
            pl.BlockSpec((_BLK // 128, 128), lambda i: (i, 0)),
        ],
        out_specs=pl.BlockSpec((_BLK, _D), lambda i: (i, 0)),
        out_shape=jax.ShapeDtypeStruct((_TOKENS, _D), jnp.float32),
        compiler_params=pltpu.CompilerParams(
            vmem_limit_bytes=128 * 1024 * 1024),
    )(flat, mask)


_NC, _NS = 2, 16
_NW = _NC * _NS
_CK = 32
_CD = 16
_RING = 4


def _pad_split(idx, chunk):
    per_w = -(-len(idx) // (_NW * chunk)) * chunk
    total = per_w * _NW
    pad = np.resize(idx[-1:], total - len(idx)) if total > len(idx) else idx[:0]
    return np.concatenate([idx, pad]).astype(np.int32).reshape(_NW, per_w // chunk, chunk)


def _kernel_sc(flat):
    mask = _dropout_mask()
    kept3 = _pad_split(np.flatnonzero(mask), _CK)
    drop3 = _pad_split(np.flatnonzero(~mask), _CD)
    nch, ndch = kept3.shape[1], drop3.shape[1]

    mesh = plsc.VectorSubcoreMesh(core_axis_name="c", subcore_axis_name="s")

    @functools.partial(
        pl.kernel,
        out_type=jax.ShapeDtypeStruct((_TOKENS, _D), jnp.float32),
        mesh=mesh,
        scratch_types=(
            [pltpu.VMEM((nch, _CK), jnp.int32),
             pltpu.VMEM((ndch, _CD), jnp.int32),
             pltpu.VMEM((_CD, _D), jnp.float32)]
            + [pltpu.VMEM((_CK, _D), jnp.float32) for _ in range(_RING)]
            + [pltpu.SemaphoreType.DMA for _ in range(2 * _RING + 1)]
        ),
    )
    def body(flat_hbm, kidx_hbm, didx_hbm, out_hbm, kidx_v, didx_v, zbuf,
             *bufs_and_sems):
        bufs = bufs_and_sems[:_RING]
        gsem = bufs_and_sems[_RING:2 * _RING]
        ssem = bufs_and_sems[2 * _RING:3 * _RING]
        zsem = bufs_and_sems[3 * _RING]
        wid = lax.axis_index("s") * _NC + lax.axis_index("c")

        pltpu.sync_copy(kidx_hbm.at[wid], kidx_v)
        pltpu.sync_copy(didx_hbm.at[wid], didx_v)

        zero = jnp.zeros((16,), jnp.float32)
        for r in range(_CD):
            for k in range(_D // 16):
                zbuf[r, pl.ds(k * 16, 16)] = zero
        zh = [pltpu.async_copy(zbuf, out_hbm.at[didx_v.at[j]], zsem)
              for j in range(ndch)]

        g = [None] * nch
        s = [None] * nch
        waited = [False] * nch
        for j in range(nch + 1):
            if j < nch:
                b = j % _RING
                if j >= _RING:
                    s[j - _RING].wait()
                    waited[j - _RING] = True
                g[j] = pltpu.async_copy(flat_hbm.at[kidx_v.at[j]], bufs[b], gsem[b])
            if j >= 1:
                i = j - 1
                g[i].wait()
                s[i] = pltpu.async_copy(bufs[i % _RING], out_hbm.at[kidx_v.at[i]],
                                        ssem[i % _RING])
        for i in range(nch):
            if not waited[i]:
                s[i].wait()
        for h in zh:
            h.wait()

    return body(flat, jnp.asarray(kept3), jnp.asarray(drop3))


_RC = 64
_LRING = 3


def _pad_split_by_range(idx, chunk, rows_w):
    per_w = [idx[(idx >= w * rows_w) & (idx < (w + 1) * rows_w)] for w in range(_NW)]
    assert all(len(p) > 0 for p in per_w)
    ndch = max(-(-len(p) // chunk) for p in per_w)
    out = np.empty((_NW, ndch * chunk), np.int32)
    for w, p in enumerate(per_w):
        out[w, :len(p)] = p
        out[w, len(p):] = p[-1]
    return out.reshape(_NW, ndch, chunk)


def _kernel_sc_linear(flat):
    mask = _dropout_mask()
    rows_w = _TOKENS // _NW
    drop3 = _pad_split_by_range(np.flatnonzero(~mask), _CD, rows_w)
    ndch = drop3.shape[1]
    nch = rows_w // _RC

    mesh = plsc.VectorSubcoreMesh(core_axis_name="c", subcore_axis_name="s")

    @functools.partial(
        pl.kernel,
        out_type=jax.ShapeDtypeStruct((_TOKENS, _D), jnp.float32),
        mesh=mesh,
        scratch_types=(
            [pltpu.VMEM((ndch, _CD), jnp.int32),
             pltpu.VMEM((_CD, _D), jnp.float32)]
            + [pltpu.VMEM((_RC, _D), jnp.float32) for _ in range(_LRING)]
            + [pltpu.SemaphoreType.DMA for _ in range(2 * _LRING + 1)]
        ),
    )
    def body(flat_hbm, didx_hbm, out_hbm, didx_v, zbuf, *bufs_and_sems):
        bufs = bufs_and_sems[:_LRING]
        gsem = bufs_and_sems[_LRING:2 * _LRING]
        ssem = bufs_and_sems[2 * _LRING:3 * _LRING]
        zsem = bufs_and_sems[3 * _LRING]
        wid = lax.axis_index("s") * _NC + lax.axis_index("c")
        base = wid * rows_w

        pltpu.sync_copy(didx_hbm.at[wid], didx_v)

        zero = jnp.zeros((16,), jnp.float32)
        for r in range(_CD):
            for k in range(_D // 16):
                zbuf[r, pl.ds(k * 16, 16)] = zero

        g = [None] * nch
        s = [None] * nch
        waited = [False] * nch
        for j in range(nch + 1):
            if j < nch:
                b = j % _LRING
                if j >= _LRING:
                    s[j - _LRING].wait()
                    waited[j - _LRING] = True
                g[j] = pltpu.async_copy(
                    flat_hbm.at[pl.ds(base + j * _RC, _RC)], bufs[b], gsem[b])
            if j >= 1:
                i = j - 1
                g[i].wait()
                s[i] = pltpu.async_copy(
                    bufs[i % _LRING], out_hbm.at[pl.ds(base + i * _RC, _RC)],
                    ssem[i % _LRING])
        for i in range(nch):
            if not waited[i]:
                s[i].wait()
        zh = [pltpu.async_copy(zbuf, out_hbm.at[didx_v.at[j]], zsem)
              for j in range(ndch)]
        for h in zh:
            h.wait()

    return body(flat, jnp.asarray(drop3))


_MCHUNK = 4096
_MIN_DEPTH = 3
_MOUT_DEPTH = 2


def _kernel_tc_manual(flat):
    mask = jnp.asarray(
        _dropout_mask().astype(np.float32).reshape(_TOKENS // 128, 128))
    n = _TOKENS // _MCHUNK
    mrows = _MCHUNK // 128

    def body(x_hbm, m_ref, o_hbm, ibufs, obufs, isems, osems):
        def copy_in(j):
            return pltpu.make_async_copy(
                x_hbm.at[pl.ds(j * _MCHUNK, _MCHUNK)],
                ibufs.at[j % _MIN_DEPTH], isems.at[j % _MIN_DEPTH])

        def copy_out(j):
            return pltpu.make_async_copy(
                obufs.at[j % _MOUT_DEPTH],
                o_hbm.at[pl.ds(j * _MCHUNK, _MCHUNK)], osems.at[j % _MOUT_DEPTH])

        for k in range(min(_MIN_DEPTH, n)):
            copy_in(k).start()
        for j in range(n):
            copy_in(j).wait()
            if j >= _MOUT_DEPTH:
                copy_out(j - _MOUT_DEPTH).wait()
            x = ibufs[j % _MIN_DEPTH].reshape(mrows, 128, _D)
            m = m_ref[pl.ds(j * mrows, mrows), :].reshape(mrows, 128, 1)
            obufs[j % _MOUT_DEPTH] = (x * m).reshape(_MCHUNK, _D)
            copy_out(j).start()
            nk = j + _MIN_DEPTH
            if nk < n:
                copy_in(nk).start()
        for j in range(max(0, n - _MOUT_DEPTH), n):
            copy_out(j).wait()

    return pl.pallas_call(
        body,
        in_specs=[
            pl.BlockSpec(memory_space=pl.ANY),
            pl.BlockSpec(memory_space=pltpu.VMEM),
        ],
        out_specs=pl.BlockSpec(memory_space=pl.ANY),
        out_shape=jax.ShapeDtypeStruct((_TOKENS, _D), jnp.float32),
        scratch_shapes=[
            pltpu.VMEM((_MIN_DEPTH, _MCHUNK, _D), jnp.float32),
            pltpu.VMEM((_MOUT_DEPTH, _MCHUNK, _D), jnp.float32),
            pltpu.SemaphoreType.DMA((_MIN_DEPTH,)),
            pltpu.SemaphoreType.DMA((_MOUT_DEPTH,)),
        ],
        compiler_params=pltpu.CompilerParams(
            vmem_limit_bytes=60 * 1024 * 1024),
    )(flat, mask)


def kernel(flat, row_starts):
    del row_starts
    return _kernel_tc_manual(flat)
